# speedup vs baseline: 2.8397x; 2.8397x over previous
import jax
import jax.numpy as jnp
from jax.experimental import pallas as pl


def _add_kernel(x_ref, pos_ref, o_ref):
    o_ref[...] = x_ref[...] + pos_ref[...]


def kernel(x, pos_table):
    batch, seq_len, d_model = x.shape
    s_blk = 512
    grid = (seq_len // s_blk, batch)
    return pl.pallas_call(
        _add_kernel,
        grid=grid,
        in_specs=[
            pl.BlockSpec((1, s_blk, d_model), lambda s, b: (b, s, 0)),
            pl.BlockSpec((s_blk, d_model), lambda s, b: (s, 0)),
        ],
        out_specs=pl.BlockSpec((1, s_blk, d_model), lambda s, b: (b, s, 0)),
        out_shape=jax.ShapeDtypeStruct(x.shape, x.dtype),
    )(x, pos_table)


# speedup vs baseline: 3.2761x; 1.1537x over previous
import jax
import jax.numpy as jnp
from jax.experimental import pallas as pl


def _add_kernel(x_ref, pos_ref, o_ref):
    o_ref[...] = x_ref[...] + pos_ref[...]


def kernel(x, pos_table):
    batch, seq_len, d_model = x.shape
    s_blk = 512
    grid = (seq_len // s_blk,)
    return pl.pallas_call(
        _add_kernel,
        grid=grid,
        in_specs=[
            pl.BlockSpec((batch, s_blk, d_model), lambda s: (0, s, 0)),
            pl.BlockSpec((s_blk, d_model), lambda s: (s, 0)),
        ],
        out_specs=pl.BlockSpec((batch, s_blk, d_model), lambda s: (0, s, 0)),
        out_shape=jax.ShapeDtypeStruct(x.shape, x.dtype),
    )(x, pos_table)
